# baseline (speedup 1.0000x reference)
import jax
import jax.numpy as jnp
from jax.experimental import pallas as pl
from jax.experimental.pallas import tpu as pltpu

_QB = 2000


def _postproc_body(obj_ref, verb_ref, sub_ref, objb_ref, scale_ref,
                   labels_ref, boxes_ref, vs_ref, scores_ref):
    x = obj_ref[0]
    c = x.shape[-1]
    m_all = jnp.max(x, axis=-1, keepdims=True)
    denom = jnp.sum(jnp.exp(x - m_all), axis=-1, keepdims=True)

    col = jax.lax.broadcasted_iota(jnp.int32, x.shape, 1)
    xm = jnp.where(col < c - 1, x, -jnp.inf)
    m_obj = jnp.max(xm, axis=-1, keepdims=True)
    label = jnp.min(jnp.where(xm == m_obj, col, c), axis=-1, keepdims=True)
    score = jnp.exp(m_obj - m_all) / denom

    vs_ref[0] = jax.nn.sigmoid(verb_ref[0]) * score

    scores_ref[0] = score
    labels_ref[0, 0] = jnp.zeros_like(label)
    labels_ref[0, 1] = label

    scale = scale_ref[0]
    for bref, slot in ((sub_ref, 0), (objb_ref, 1)):
        bx = bref[0]
        cxy = bx[:, 0:2]
        half = bx[:, 2:4] * 0.5
        xyxy = jnp.concatenate([cxy - half, cxy + half], axis=-1)
        boxes_ref[0, slot] = xyxy * scale


def kernel(pred_obj_logits, pred_verb_logits, pred_sub_boxes, pred_obj_boxes, target_sizes):
    B, Q, C = pred_obj_logits.shape
    V = pred_verb_logits.shape[-1]
    nq = Q // _QB

    img_h = target_sizes[:, 0].astype(jnp.float32)
    img_w = target_sizes[:, 1].astype(jnp.float32)
    scale = jnp.stack([img_w, img_h, img_w, img_h], axis=1).reshape(B, 1, 4)

    lab4, box4, vs, sc3 = pl.pallas_call(
        _postproc_body,
        grid=(B, nq),
        in_specs=[
            pl.BlockSpec((1, _QB, C), lambda b, q: (b, q, 0)),
            pl.BlockSpec((1, _QB, V), lambda b, q: (b, q, 0)),
            pl.BlockSpec((1, _QB, 4), lambda b, q: (b, q, 0)),
            pl.BlockSpec((1, _QB, 4), lambda b, q: (b, q, 0)),
            pl.BlockSpec((1, 1, 4), lambda b, q: (b, 0, 0)),
        ],
        out_specs=[
            pl.BlockSpec((1, 2, _QB, 1), lambda b, q: (b, 0, q, 0)),
            pl.BlockSpec((1, 2, _QB, 4), lambda b, q: (b, 0, q, 0)),
            pl.BlockSpec((1, _QB, V), lambda b, q: (b, q, 0)),
            pl.BlockSpec((1, _QB, 1), lambda b, q: (b, q, 0)),
        ],
        out_shape=[
            jax.ShapeDtypeStruct((B, 2, Q, 1), jnp.int32),
            jax.ShapeDtypeStruct((B, 2, Q, 4), jnp.float32),
            jax.ShapeDtypeStruct((B, Q, V), jnp.float32),
            jax.ShapeDtypeStruct((B, Q, 1), jnp.float32),
        ],
        compiler_params=pltpu.CompilerParams(
            dimension_semantics=("parallel", "parallel")),
    )(pred_obj_logits, pred_verb_logits, pred_sub_boxes, pred_obj_boxes, scale)

    labels = lab4.reshape(B, 2 * Q)
    boxes = box4.reshape(B, 2 * Q, 4)
    obj_scores = sc3.reshape(B, Q)
    ids = jnp.arange(2 * Q)
    return (labels, boxes, vs, pred_verb_logits, ids[:Q], ids[Q:], obj_scores)
